# Initial kernel scaffold; baseline (speedup 1.0000x reference)
import jax
import jax.numpy as jnp
from jax.experimental import pallas as pl


def kernel(x, emb_table, pos_table):
    raise NotImplementedError("write your pallas kernel here")


# speedup vs baseline: 3.0343x; 3.0343x over previous
import functools

import jax
import jax.numpy as jnp
from jax import lax
from jax.experimental import pallas as pl
from jax.experimental.pallas import tpu as pltpu
from jax.experimental.pallas import tpu_sc as plsc

NC = 2
NS = 16
NW = NC * NS
LANES = 16
CHUNK = 128


def _build_sc_call(n_rows, h_dim):
    assert h_dim == LANES
    assert n_rows % (NW * CHUNK) == 0
    rows_per_w = n_rows // NW
    n_chunks = rows_per_w // CHUNK
    mesh = plsc.VectorSubcoreMesh(core_axis_name="c", subcore_axis_name="s")

    @functools.partial(
        pl.kernel,
        mesh=mesh,
        out_type=jax.ShapeDtypeStruct((n_rows, h_dim), jnp.float32),
        compiler_params=pltpu.CompilerParams(
            needs_layout_passes=False, use_tc_tiling_on_sc=False),
        scratch_types=[
            pltpu.VMEM((n_chunks, CHUNK), jnp.int32),
            pltpu.VMEM((CHUNK, LANES), jnp.float32),
            pltpu.VMEM((CHUNK, LANES), jnp.float32),
            pltpu.VMEM((LANES * LANES,), jnp.float32),
            pltpu.SemaphoreType.DMA,
        ],
    )
    def sc_encode(x_hbm, emb_hbm, pos_hbm, out_hbm, idx_v, e_buf, o_buf,
                  pos_v, sem):
        wid = lax.axis_index("s") * NC + lax.axis_index("c")
        pltpu.sync_copy(x_hbm.at[wid], idx_v)
        pltpu.sync_copy(pos_hbm, pos_v)
        riota = lax.iota(jnp.int32, LANES)

        def chunk_body(j, carry):
            pltpu.async_copy(emb_hbm.at[idx_v.at[j]], e_buf, sem).wait()
            for r in range(CHUNK):
                row = e_buf[r]
                m = jnp.max(row)
                amax = plsc.all_reduce_ffs(row == m)
                pos_row = plsc.load_gather(pos_v, [amax * LANES + riota])
                o_buf[r] = row + pos_row
            row0 = wid * rows_per_w + j * CHUNK
            pltpu.sync_copy(o_buf, out_hbm.at[pl.ds(row0, CHUNK)])
            return carry

        lax.fori_loop(0, n_chunks, chunk_body, 0)

    return sc_encode


def kernel(x, emb_table, pos_table):
    b, l = x.shape
    _, h_dim = emb_table.shape
    n_rows = b * l
    rows_per_w = n_rows // NW
    xf = x.reshape(-1).astype(jnp.int32).reshape(NW, rows_per_w // CHUNK, CHUNK)
    call = _build_sc_call(n_rows, h_dim)
    out = call(xf, emb_table, pos_table.reshape(-1))
    return out.reshape(b, l, h_dim)


# speedup vs baseline: 3.0344x; 1.0000x over previous
import functools

import jax
import jax.numpy as jnp
from jax import lax
from jax.experimental import pallas as pl
from jax.experimental.pallas import tpu as pltpu
from jax.experimental.pallas import tpu_sc as plsc

NC = 2
NS = 16
NW = NC * NS
LANES = 16
CHUNK = 128


def _build_sc_call(n_rows, h_dim):
    assert h_dim == LANES
    assert n_rows % (NW * CHUNK) == 0
    rows_per_w = n_rows // NW
    n_chunks = rows_per_w // CHUNK
    mesh = plsc.VectorSubcoreMesh(core_axis_name="c", subcore_axis_name="s")

    @functools.partial(
        pl.kernel,
        mesh=mesh,
        out_type=jax.ShapeDtypeStruct((n_rows, h_dim), jnp.float32),
        compiler_params=pltpu.CompilerParams(
            needs_layout_passes=False, use_tc_tiling_on_sc=False),
        scratch_types=[
            pltpu.VMEM((n_chunks, CHUNK), jnp.int32),
            pltpu.VMEM((CHUNK, LANES), jnp.float32),
            pltpu.VMEM((CHUNK, LANES), jnp.float32),
            pltpu.VMEM((CHUNK, LANES), jnp.float32),
            pltpu.VMEM((CHUNK, LANES), jnp.float32),
            pltpu.VMEM((LANES * LANES,), jnp.float32),
            pltpu.SemaphoreType.DMA,
            pltpu.SemaphoreType.DMA,
            pltpu.SemaphoreType.DMA,
            pltpu.SemaphoreType.DMA,
        ],
    )
    def sc_encode(x_hbm, emb_hbm, pos_hbm, out_hbm, idx_v, e_buf0, e_buf1,
                  o_buf0, o_buf1, pos_v, gsem0, gsem1, ssem0, ssem1):
        wid = lax.axis_index("s") * NC + lax.axis_index("c")
        e_bufs, o_bufs = (e_buf0, e_buf1), (o_buf0, o_buf1)
        gsems, ssems = (gsem0, gsem1), (ssem0, ssem1)
        pltpu.sync_copy(x_hbm.at[wid], idx_v)
        pltpu.sync_copy(pos_hbm, pos_v)
        riota = lax.iota(jnp.int32, LANES)
        out_base = wid * rows_per_w
        nbuf = 2

        for b in range(nbuf):
            pltpu.async_copy(emb_hbm.at[idx_v.at[b]], e_bufs[b], gsems[b])

        def chunk_body(g, carry):
            for b in range(nbuf):
                j = g * nbuf + b
                pltpu.make_async_copy(
                    emb_hbm.at[idx_v.at[j]], e_bufs[b], gsems[b]).wait()

                @pl.when(g > 0)
                def _():
                    prow0 = out_base + (j - nbuf) * CHUNK
                    pltpu.make_async_copy(
                        o_bufs[b], out_hbm.at[pl.ds(prow0, CHUNK)],
                        ssems[b]).wait()

                for r in range(CHUNK):
                    row = e_bufs[b][r]
                    m = jnp.max(row)
                    amax = plsc.all_reduce_ffs(row == m)
                    pos_row = plsc.load_gather(pos_v, [amax * LANES + riota])
                    o_bufs[b][r] = row + pos_row

                pltpu.async_copy(
                    o_bufs[b], out_hbm.at[pl.ds(out_base + j * CHUNK, CHUNK)],
                    ssems[b])

                @pl.when(j + nbuf < n_chunks)
                def _():
                    pltpu.async_copy(
                        emb_hbm.at[idx_v.at[j + nbuf]], e_bufs[b], gsems[b])
            return carry

        lax.fori_loop(0, n_chunks // nbuf, chunk_body, 0)

        for b in range(nbuf):
            j = n_chunks - nbuf + b
            pltpu.make_async_copy(
                o_bufs[b], out_hbm.at[pl.ds(out_base + j * CHUNK, CHUNK)],
                ssems[b]).wait()

    return sc_encode


def kernel(x, emb_table, pos_table):
    b, l = x.shape
    _, h_dim = emb_table.shape
    n_rows = b * l
    rows_per_w = n_rows // NW
    xf = x.reshape(-1).astype(jnp.int32).reshape(NW, rows_per_w // CHUNK, CHUNK)
    call = _build_sc_call(n_rows, h_dim)
    out = call(xf, emb_table, pos_table.reshape(-1))
    return out.reshape(b, l, h_dim)


# speedup vs baseline: 3.6071x; 1.1887x over previous
import functools

import jax
import jax.numpy as jnp
from jax import lax
from jax.experimental import pallas as pl
from jax.experimental.pallas import tpu as pltpu
from jax.experimental.pallas import tpu_sc as plsc

NC = 2
NS = 16
NW = NC * NS
LANES = 16
CHUNK = 128


def _build_sc_call(n_rows, h_dim):
    assert h_dim == LANES
    assert n_rows % (NW * CHUNK) == 0
    rows_per_w = n_rows // NW
    n_chunks = rows_per_w // CHUNK
    mesh = plsc.VectorSubcoreMesh(core_axis_name="c", subcore_axis_name="s")

    @functools.partial(
        pl.kernel,
        mesh=mesh,
        out_type=jax.ShapeDtypeStruct((n_rows, h_dim), jnp.float32),
        compiler_params=pltpu.CompilerParams(
            needs_layout_passes=False, use_tc_tiling_on_sc=False),
        scratch_types=[
            pltpu.VMEM((n_chunks, CHUNK), jnp.int32),
            pltpu.VMEM((CHUNK, LANES), jnp.float32),
            pltpu.VMEM((CHUNK, LANES), jnp.float32),
            pltpu.VMEM((CHUNK, LANES), jnp.float32),
            pltpu.VMEM((CHUNK, LANES), jnp.float32),
            pltpu.VMEM((LANES * LANES,), jnp.float32),
            pltpu.SemaphoreType.DMA,
            pltpu.SemaphoreType.DMA,
            pltpu.SemaphoreType.DMA,
            pltpu.SemaphoreType.DMA,
        ],
    )
    def sc_encode(x_hbm, emb_hbm, pos_hbm, out_hbm, idx_v, e_buf0, e_buf1,
                  o_buf0, o_buf1, pos_v, gsem0, gsem1, ssem0, ssem1):
        wid = lax.axis_index("s") * NC + lax.axis_index("c")
        e_bufs, o_bufs = (e_buf0, e_buf1), (o_buf0, o_buf1)
        gsems, ssems = (gsem0, gsem1), (ssem0, ssem1)
        pltpu.sync_copy(x_hbm.at[wid], idx_v)
        pltpu.sync_copy(pos_hbm, pos_v)
        riota = lax.iota(jnp.int32, LANES)
        out_base = wid * rows_per_w
        nbuf = 2

        for b in range(nbuf):
            pltpu.async_copy(emb_hbm.at[idx_v.at[b]], e_bufs[b], gsems[b])

        def chunk_body(g, carry):
            for b in range(nbuf):
                j = g * nbuf + b
                pltpu.make_async_copy(
                    emb_hbm.at[idx_v.at[j]], e_bufs[b], gsems[b]).wait()

                @pl.when(g > 0)
                def _():
                    prow0 = out_base + (j - nbuf) * CHUNK
                    pltpu.make_async_copy(
                        o_bufs[b], out_hbm.at[pl.ds(prow0, CHUNK)],
                        ssems[b]).wait()

                e_buf, o_buf = e_bufs[b], o_bufs[b]

                @plsc.parallel_loop(0, CHUNK, unroll=8)
                def _(r):
                    row = e_buf[r]
                    m = jnp.max(row)
                    amax = plsc.all_reduce_ffs(row == m)
                    pos_row = plsc.load_gather(pos_v, [amax * LANES + riota])
                    o_buf[r] = row + pos_row

                pltpu.async_copy(
                    o_bufs[b], out_hbm.at[pl.ds(out_base + j * CHUNK, CHUNK)],
                    ssems[b])

                @pl.when(j + nbuf < n_chunks)
                def _():
                    pltpu.async_copy(
                        emb_hbm.at[idx_v.at[j + nbuf]], e_bufs[b], gsems[b])
            return carry

        lax.fori_loop(0, n_chunks // nbuf, chunk_body, 0)

        for b in range(nbuf):
            j = n_chunks - nbuf + b
            pltpu.make_async_copy(
                o_bufs[b], out_hbm.at[pl.ds(out_base + j * CHUNK, CHUNK)],
                ssems[b]).wait()

    return sc_encode


def kernel(x, emb_table, pos_table):
    b, l = x.shape
    _, h_dim = emb_table.shape
    n_rows = b * l
    rows_per_w = n_rows // NW
    xf = x.reshape(-1).astype(jnp.int32).reshape(NW, rows_per_w // CHUNK, CHUNK)
    call = _build_sc_call(n_rows, h_dim)
    out = call(xf, emb_table, pos_table.reshape(-1))
    return out.reshape(b, l, h_dim)
